# baseline (speedup 1.0000x reference)
import functools
import math

import jax
import jax.numpy as jnp
from jax import lax
from jax.experimental import pallas as pl
from jax.experimental.pallas import tpu as pltpu
from jax.experimental.pallas import tpu_sc as plsc

BATCH = 16384
EMBED = 64
FEAT = 64
HIDDEN = 256
KDIM = EMBED + EMBED + FEAT
PAIR = 2 * EMBED
NROWS = 1000000
NPAIR = NROWS // 2

NC = 2
NS = 16
NW = NC * NS
B_PER_W = BATCH // NW
CHUNK = 128
NCHUNK = B_PER_W // CHUNK

TBLK = 12800
NSPLIT = 39
SPLIT = NSPLIT * TBLK
NPAIR2 = NROWS - SPLIT
TGRID = -(-NPAIR2 // TBLK)


def _transpose_body(hi_ref, lo_ref, dst_ref):
    dst_ref[:, 0:EMBED] = hi_ref[...].T
    dst_ref[:, EMBED:PAIR] = lo_ref[...].T


def _transpose_pairs(tabT):
    return pl.pallas_call(
        _transpose_body,
        grid=(TGRID,),
        in_specs=[pl.BlockSpec((EMBED, TBLK), lambda i: (0, i + NSPLIT)),
                  pl.BlockSpec((EMBED, TBLK), lambda i: (0, i))],
        out_specs=pl.BlockSpec((TBLK, PAIR), lambda i: (i, 0)),
        out_shape=jax.ShapeDtypeStruct((NPAIR2, PAIR), jnp.float32),
    )(tabT, tabT)


SC_TILES = SPLIT // 128
TPW = -(-SC_TILES // NW)


def _sc_transpose_body(tabT_hbm, out_hbm, slab_hi, slab_lo, otile, sem):
    wid = lax.axis_index("s") * NC + lax.axis_index("c")
    t0 = wid * TPW
    t1 = jnp.minimum((wid + 1) * TPW, SC_TILES)
    iota = lax.iota(jnp.int32, 16)

    def per_tile(t, carry):
        col_lo = pl.multiple_of(t * 128, 128)
        col_hi = pl.multiple_of(SPLIT + t * 128, 128)
        chi = pltpu.async_copy(tabT_hbm.at[:, pl.ds(col_hi, 128)], slab_hi, sem)
        clo = pltpu.async_copy(tabT_hbm.at[:, pl.ds(col_lo, 128)], slab_lo, sem)
        chi.wait()
        clo.wait()

        def per_feat(f, c2):
            for j in range(8):
                vh = slab_hi[f, pl.ds(16 * j, 16)]
                vl = slab_lo[f, pl.ds(16 * j, 16)]
                rows16 = iota + (16 * j)
                plsc.store_scatter(otile, [rows16, jnp.full((16,), f, jnp.int32)], vh)
                plsc.store_scatter(otile, [rows16, jnp.full((16,), EMBED + f, jnp.int32)], vl)
            return c2

        lax.fori_loop(0, EMBED, per_feat, 0)
        pltpu.sync_copy(otile, out_hbm.at[pl.ds(pl.multiple_of(t * 128, 128), 128)])
        return carry

    lax.fori_loop(t0, t1, per_tile, 0)


@functools.lru_cache(maxsize=None)
def _sc_transpose():
    return pl.kernel(
        _sc_transpose_body,
        out_type=jax.ShapeDtypeStruct((NPAIR2, PAIR), jnp.float32),
        mesh=plsc.VectorSubcoreMesh(core_axis_name="c", subcore_axis_name="s",
                                    num_cores=NC, num_subcores=NS),
        scratch_types=[
            pltpu.VMEM((EMBED, 128), jnp.float32),
            pltpu.VMEM((EMBED, 128), jnp.float32),
            pltpu.VMEM((128, PAIR), jnp.float32),
            pltpu.SemaphoreType.DMA,
        ],
        compiler_params=pltpu.CompilerParams(use_tc_tiling_on_sc=True,
                                             needs_layout_passes=False),
    )


def _tail_body(prev_ref, hi_ref, lo_ref, dst_ref):
    dst_ref[:, 0:EMBED] = hi_ref[...].T
    dst_ref[:, EMBED:PAIR] = lo_ref[...].T


def _tail_fixup(sc_out, tabT):
    return pl.pallas_call(
        _tail_body,
        grid=(1,),
        in_specs=[
            pl.BlockSpec(memory_space=pl.ANY),
            pl.BlockSpec((EMBED, TBLK), lambda i: (0, 2 * NSPLIT)),
            pl.BlockSpec((EMBED, TBLK), lambda i: (0, NSPLIT)),
        ],
        out_specs=pl.BlockSpec((TBLK, PAIR), lambda i: (NSPLIT, 0)),
        out_shape=jax.ShapeDtypeStruct((NPAIR2, PAIR), jnp.float32),
        input_output_aliases={0: 0},
    )(sc_out, tabT, tabT)


def _gather_body(idx_hbm, tab_hbm, out_hbm, idx_v, rows, sem):
    wid = lax.axis_index("s") * NC + lax.axis_index("c")
    base = wid * B_PER_W
    pltpu.sync_copy(idx_hbm.at[wid], idx_v)
    copies = []
    for j in range(NCHUNK):
        copies.append(pltpu.async_copy(
            tab_hbm.at[idx_v.at[j]], rows.at[pl.ds(j * CHUNK, CHUNK)], sem))
    for c in copies:
        c.wait()
    pltpu.sync_copy(rows, out_hbm.at[pl.ds(base, B_PER_W)])


@functools.lru_cache(maxsize=None)
def _sc_gather():
    return pl.kernel(
        _gather_body,
        out_type=jax.ShapeDtypeStruct((BATCH, PAIR), jnp.float32),
        mesh=plsc.VectorSubcoreMesh(core_axis_name="c", subcore_axis_name="s",
                                    num_cores=NC, num_subcores=NS),
        scratch_types=[
            pltpu.VMEM((NCHUNK, CHUNK), jnp.int32),
            pltpu.VMEM((B_PER_W, PAIR), jnp.float32),
            pltpu.SemaphoreType.DMA,
        ],
        compiler_params=pltpu.CompilerParams(use_tc_tiling_on_sc=True),
    )


ROWS_BLK = 2048
GRID = BATCH // ROWS_BLK


def _dot_t(lhs_t, rhs):
    return lax.dot_general(lhs_t, rhs, (((0,), (0,)), ((), ())),
                           preferred_element_type=jnp.float32)


def _mlp_body(u2_ref, i2_ref, ft_ref, su_ref, si_ref, w1_ref, b1_ref,
              wrp_ref, brp_ref, rat_ref, play_ref):
    u2 = u2_ref[...]
    i2 = i2_ref[...]
    su = su_ref[...]
    si = si_ref[...]
    u = u2[:, 0:EMBED] + (u2[:, EMBED:PAIR] - u2[:, 0:EMBED]) * su
    i = i2[:, 0:EMBED] + (i2[:, EMBED:PAIR] - i2[:, 0:EMBED]) * si
    x = (jnp.dot(u, w1_ref[0:EMBED, :], preferred_element_type=jnp.float32)
         + jnp.dot(i, w1_ref[EMBED:2 * EMBED, :],
                   preferred_element_type=jnp.float32)
         + _dot_t(ft_ref[...], w1_ref[2 * EMBED:KDIM, :])
         + b1_ref[...])
    h = 0.5 * x * (1.0 + lax.erf(x * (1.0 / math.sqrt(2.0))))
    o = jnp.dot(h, wrp_ref[...], preferred_element_type=jnp.float32) + brp_ref[...]
    rat_ref[...] = jax.nn.sigmoid(o[:, 0:1])
    play_ref[...] = jnp.maximum(o[:, 1:2], 0.0)


def _mlp(u2_rows, i2_rows, fT, sel_u, sel_i, W1, b1, Wrp, brp,
         interpret=False):
    return pl.pallas_call(
        _mlp_body,
        grid=(GRID,),
        in_specs=[
            pl.BlockSpec((ROWS_BLK, PAIR), lambda i: (i, 0)),
            pl.BlockSpec((ROWS_BLK, PAIR), lambda i: (i, 0)),
            pl.BlockSpec((FEAT, ROWS_BLK), lambda i: (0, i)),
            pl.BlockSpec((ROWS_BLK, 1), lambda i: (i, 0)),
            pl.BlockSpec((ROWS_BLK, 1), lambda i: (i, 0)),
            pl.BlockSpec((KDIM, HIDDEN), lambda i: (0, 0)),
            pl.BlockSpec((1, HIDDEN), lambda i: (0, 0)),
            pl.BlockSpec((HIDDEN, 2), lambda i: (0, 0)),
            pl.BlockSpec((1, 2), lambda i: (0, 0)),
        ],
        out_specs=[
            pl.BlockSpec((ROWS_BLK, 1), lambda i: (i, 0)),
            pl.BlockSpec((ROWS_BLK, 1), lambda i: (i, 0)),
        ],
        out_shape=[
            jax.ShapeDtypeStruct((BATCH, 1), jnp.float32),
            jax.ShapeDtypeStruct((BATCH, 1), jnp.float32),
        ],
        interpret=interpret,
    )(u2_rows, i2_rows, fT, sel_u, sel_i, W1, b1, Wrp, brp)


def kernel(user_input, item_input, feature_input, user_emb, item_emb,
           W1, b1, Wr, br, Wp, bp):
    u2tab = _transpose_pairs(user_emb.T)
    iT = item_emb.T
    i2tab = _tail_fixup(_sc_transpose()(iT), iT)
    uidx = jnp.where(user_input >= SPLIT, user_input - SPLIT,
                     user_input).reshape(NW, NCHUNK, CHUNK)
    iidx = jnp.where(item_input >= SPLIT, item_input - SPLIT,
                     item_input).reshape(NW, NCHUNK, CHUNK)
    sel_u = (user_input < SPLIT).astype(jnp.float32).reshape(BATCH, 1)
    sel_i = (item_input < SPLIT).astype(jnp.float32).reshape(BATCH, 1)
    gather = _sc_gather()
    u2_rows = gather(uidx, u2tab)
    i2_rows = gather(iidx, i2tab)
    fT = feature_input.T
    Wrp = jnp.concatenate([Wr, Wp], axis=1)
    brp = jnp.concatenate([br, bp]).reshape(1, 2)
    rating, playtime = _mlp(u2_rows, i2_rows, fT, sel_u, sel_i,
                            W1, b1.reshape(1, HIDDEN), Wrp, brp)
    return (rating, playtime)


# speedup vs baseline: 2.9667x; 2.9667x over previous
import functools
import math

import jax
import jax.numpy as jnp
from jax import lax
from jax.experimental import pallas as pl
from jax.experimental.pallas import tpu as pltpu
from jax.experimental.pallas import tpu_sc as plsc

BATCH = 16384
EMBED = 64
FEAT = 64
HIDDEN = 256
KDIM = EMBED + EMBED + FEAT
PAIR = 2 * EMBED
NROWS = 1000000
NPAIR = NROWS // 2

NC = 2
NS = 16
NW = NC * NS
B_PER_W = BATCH // NW
CHUNK = 128
NCHUNK = B_PER_W // CHUNK

TBLK = 25600
NSPLIT = 19
SPLIT = NSPLIT * TBLK
NPAIR2 = NROWS - SPLIT
TGRID = -(-NPAIR2 // TBLK)


def _transpose_body(hi_ref, lo_ref, dst_ref):
    dst_ref[:, 0:EMBED] = hi_ref[...].T
    dst_ref[:, EMBED:PAIR] = lo_ref[...].T


def _transpose_pairs(tabT):
    return pl.pallas_call(
        _transpose_body,
        grid=(TGRID,),
        in_specs=[pl.BlockSpec((EMBED, TBLK), lambda i: (0, i + NSPLIT)),
                  pl.BlockSpec((EMBED, TBLK), lambda i: (0, i))],
        out_specs=pl.BlockSpec((TBLK, PAIR), lambda i: (i, 0)),
        out_shape=jax.ShapeDtypeStruct((NPAIR2, PAIR), jnp.float32),
        compiler_params=pltpu.CompilerParams(
            vmem_limit_bytes=110 * 1024 * 1024),
    )(tabT, tabT)


SC_TILES = SPLIT // 128
TPW = -(-SC_TILES // NW)


def _sc_transpose_body(tabT_hbm, out_hbm, slab_hi, slab_lo, otile, sem):
    wid = lax.axis_index("s") * NC + lax.axis_index("c")
    t0 = wid * TPW
    t1 = jnp.minimum((wid + 1) * TPW, SC_TILES)
    iota = lax.iota(jnp.int32, 16)

    def per_tile(t, carry):
        col_lo = pl.multiple_of(t * 128, 128)
        col_hi = pl.multiple_of(SPLIT + t * 128, 128)
        chi = pltpu.async_copy(tabT_hbm.at[:, pl.ds(col_hi, 128)], slab_hi, sem)
        clo = pltpu.async_copy(tabT_hbm.at[:, pl.ds(col_lo, 128)], slab_lo, sem)
        chi.wait()
        clo.wait()

        def per_feat(f, c2):
            for j in range(8):
                vh = slab_hi[f, pl.ds(16 * j, 16)]
                vl = slab_lo[f, pl.ds(16 * j, 16)]
                rows16 = iota + (16 * j)
                plsc.store_scatter(otile, [rows16, jnp.full((16,), f, jnp.int32)], vh)
                plsc.store_scatter(otile, [rows16, jnp.full((16,), EMBED + f, jnp.int32)], vl)
            return c2

        lax.fori_loop(0, EMBED, per_feat, 0)
        pltpu.sync_copy(otile, out_hbm.at[pl.ds(pl.multiple_of(t * 128, 128), 128)])
        return carry

    lax.fori_loop(t0, t1, per_tile, 0)


@functools.lru_cache(maxsize=None)
def _sc_transpose():
    return pl.kernel(
        _sc_transpose_body,
        out_type=jax.ShapeDtypeStruct((NPAIR2, PAIR), jnp.float32),
        mesh=plsc.VectorSubcoreMesh(core_axis_name="c", subcore_axis_name="s",
                                    num_cores=NC, num_subcores=NS),
        scratch_types=[
            pltpu.VMEM((EMBED, 128), jnp.float32),
            pltpu.VMEM((EMBED, 128), jnp.float32),
            pltpu.VMEM((128, PAIR), jnp.float32),
            pltpu.SemaphoreType.DMA,
        ],
        compiler_params=pltpu.CompilerParams(use_tc_tiling_on_sc=True,
                                             needs_layout_passes=False),
    )


def _tail_body(prev_ref, hi_ref, lo_ref, dst_ref):
    dst_ref[:, 0:EMBED] = hi_ref[...].T
    dst_ref[:, EMBED:PAIR] = lo_ref[...].T


def _tail_fixup(sc_out, tabT):
    return pl.pallas_call(
        _tail_body,
        grid=(1,),
        in_specs=[
            pl.BlockSpec(memory_space=pl.ANY),
            pl.BlockSpec((EMBED, TBLK), lambda i: (0, 2 * NSPLIT)),
            pl.BlockSpec((EMBED, TBLK), lambda i: (0, NSPLIT)),
        ],
        out_specs=pl.BlockSpec((TBLK, PAIR), lambda i: (NSPLIT, 0)),
        out_shape=jax.ShapeDtypeStruct((NPAIR2, PAIR), jnp.float32),
        input_output_aliases={0: 0},
    )(sc_out, tabT, tabT)


def _gather_body(idx_hbm, tab_hbm, out_hbm, idx_v, rows, sem):
    wid = lax.axis_index("s") * NC + lax.axis_index("c")
    base = wid * B_PER_W
    pltpu.sync_copy(idx_hbm.at[wid], idx_v)
    copies = []
    for j in range(NCHUNK):
        copies.append(pltpu.async_copy(
            tab_hbm.at[idx_v.at[j]], rows.at[pl.ds(j * CHUNK, CHUNK)], sem))
    for c in copies:
        c.wait()
    pltpu.sync_copy(rows, out_hbm.at[pl.ds(base, B_PER_W)])


@functools.lru_cache(maxsize=None)
def _sc_gather():
    return pl.kernel(
        _gather_body,
        out_type=jax.ShapeDtypeStruct((BATCH, PAIR), jnp.float32),
        mesh=plsc.VectorSubcoreMesh(core_axis_name="c", subcore_axis_name="s",
                                    num_cores=NC, num_subcores=NS),
        scratch_types=[
            pltpu.VMEM((NCHUNK, CHUNK), jnp.int32),
            pltpu.VMEM((B_PER_W, PAIR), jnp.float32),
            pltpu.SemaphoreType.DMA,
        ],
        compiler_params=pltpu.CompilerParams(use_tc_tiling_on_sc=True),
    )


ROWS_BLK = 2048
GRID = BATCH // ROWS_BLK


def _dot_t(lhs_t, rhs):
    return lax.dot_general(lhs_t, rhs, (((0,), (0,)), ((), ())),
                           preferred_element_type=jnp.float32)


def _mlp_body(u2_ref, i2_ref, ft_ref, su_ref, si_ref, w1_ref, b1_ref,
              wrp_ref, brp_ref, rat_ref, play_ref):
    u2 = u2_ref[...]
    i2 = i2_ref[...]
    su = su_ref[...]
    si = si_ref[...]
    u = u2[:, 0:EMBED] + (u2[:, EMBED:PAIR] - u2[:, 0:EMBED]) * su
    i = i2[:, 0:EMBED] + (i2[:, EMBED:PAIR] - i2[:, 0:EMBED]) * si
    x = (jnp.dot(u, w1_ref[0:EMBED, :], preferred_element_type=jnp.float32)
         + jnp.dot(i, w1_ref[EMBED:2 * EMBED, :],
                   preferred_element_type=jnp.float32)
         + _dot_t(ft_ref[...], w1_ref[2 * EMBED:KDIM, :])
         + b1_ref[...])
    h = 0.5 * x * (1.0 + lax.erf(x * (1.0 / math.sqrt(2.0))))
    o = jnp.dot(h, wrp_ref[...], preferred_element_type=jnp.float32) + brp_ref[...]
    rat_ref[...] = jax.nn.sigmoid(o[:, 0:1])
    play_ref[...] = jnp.maximum(o[:, 1:2], 0.0)


def _mlp(u2_rows, i2_rows, fT, sel_u, sel_i, W1, b1, Wrp, brp,
         interpret=False):
    return pl.pallas_call(
        _mlp_body,
        grid=(GRID,),
        in_specs=[
            pl.BlockSpec((ROWS_BLK, PAIR), lambda i: (i, 0)),
            pl.BlockSpec((ROWS_BLK, PAIR), lambda i: (i, 0)),
            pl.BlockSpec((FEAT, ROWS_BLK), lambda i: (0, i)),
            pl.BlockSpec((ROWS_BLK, 1), lambda i: (i, 0)),
            pl.BlockSpec((ROWS_BLK, 1), lambda i: (i, 0)),
            pl.BlockSpec((KDIM, HIDDEN), lambda i: (0, 0)),
            pl.BlockSpec((1, HIDDEN), lambda i: (0, 0)),
            pl.BlockSpec((HIDDEN, 2), lambda i: (0, 0)),
            pl.BlockSpec((1, 2), lambda i: (0, 0)),
        ],
        out_specs=[
            pl.BlockSpec((ROWS_BLK, 1), lambda i: (i, 0)),
            pl.BlockSpec((ROWS_BLK, 1), lambda i: (i, 0)),
        ],
        out_shape=[
            jax.ShapeDtypeStruct((BATCH, 1), jnp.float32),
            jax.ShapeDtypeStruct((BATCH, 1), jnp.float32),
        ],
        interpret=interpret,
    )(u2_rows, i2_rows, fT, sel_u, sel_i, W1, b1, Wrp, brp)


def kernel(user_input, item_input, feature_input, user_emb, item_emb,
           W1, b1, Wr, br, Wp, bp):
    u2tab = _transpose_pairs(user_emb.T)
    i2tab = _transpose_pairs(item_emb.T)
    uidx = jnp.where(user_input >= SPLIT, user_input - SPLIT,
                     user_input).reshape(NW, NCHUNK, CHUNK)
    iidx = jnp.where(item_input >= SPLIT, item_input - SPLIT,
                     item_input).reshape(NW, NCHUNK, CHUNK)
    sel_u = (user_input < SPLIT).astype(jnp.float32).reshape(BATCH, 1)
    sel_i = (item_input < SPLIT).astype(jnp.float32).reshape(BATCH, 1)
    gather = _sc_gather()
    u2_rows = gather(uidx, u2tab)
    i2_rows = gather(iidx, i2tab)
    fT = feature_input.T
    Wrp = jnp.concatenate([Wr, Wp], axis=1)
    brp = jnp.concatenate([br, bp]).reshape(1, 2)
    rating, playtime = _mlp(u2_rows, i2_rows, fT, sel_u, sel_i,
                            W1, b1.reshape(1, HIDDEN), Wrp, brp)
    return (rating, playtime)


# speedup vs baseline: 3.2076x; 1.0812x over previous
import functools
import math

import jax
import jax.numpy as jnp
from jax import lax
from jax.experimental import pallas as pl
from jax.experimental.pallas import tpu as pltpu
from jax.experimental.pallas import tpu_sc as plsc

BATCH = 16384
EMBED = 64
FEAT = 64
HIDDEN = 256
KDIM = EMBED + EMBED + FEAT
PAIR = 2 * EMBED
NROWS = 1000000
NPAIR = NROWS // 2

NC = 2
NS = 16
NW = NC * NS
B_PER_W = BATCH // NW
CHUNK = 128
NCHUNK = B_PER_W // CHUNK

TBLK = 12800
NSPLIT = 39
SPLIT = NSPLIT * TBLK
NPAIR2 = NROWS - SPLIT
TGRID = -(-NPAIR2 // TBLK)


def _transpose_body(hi_ref, lo_ref, dst_ref):
    dst_ref[:, 0:EMBED] = hi_ref[...].T
    dst_ref[:, EMBED:PAIR] = lo_ref[...].T


def _transpose_pairs(tabT):
    return pl.pallas_call(
        _transpose_body,
        grid=(TGRID,),
        in_specs=[pl.BlockSpec((EMBED, TBLK), lambda i: (0, i + NSPLIT)),
                  pl.BlockSpec((EMBED, TBLK), lambda i: (0, i))],
        out_specs=pl.BlockSpec((TBLK, PAIR), lambda i: (i, 0)),
        out_shape=jax.ShapeDtypeStruct((NPAIR2, PAIR), jnp.float32),
    )(tabT, tabT)


QUART = 19 * TBLK
NQUAD = NROWS - 3 * QUART
QGRID = -(-NQUAD // TBLK)


def _pack2(a, b):
    ai = lax.bitcast_convert_type(a, jnp.int32)
    bi = lax.bitcast_convert_type(b, jnp.int32)
    w = (ai & jnp.int32(-65536)) | lax.shift_right_logical(bi, 16)
    return lax.bitcast_convert_type(w, jnp.float32)


def _transpose_quad_body(q0_ref, q1_ref, q2_ref, q3_ref, dst_ref):
    dst_ref[:, 0:EMBED] = _pack2(q0_ref[...].T, q1_ref[...].T)
    dst_ref[:, EMBED:PAIR] = _pack2(q2_ref[...].T, q3_ref[...].T)


def _transpose_quad(tabT):
    return pl.pallas_call(
        _transpose_quad_body,
        grid=(QGRID,),
        in_specs=[pl.BlockSpec((EMBED, TBLK), lambda i: (0, i)),
                  pl.BlockSpec((EMBED, TBLK), lambda i: (0, i + 19)),
                  pl.BlockSpec((EMBED, TBLK), lambda i: (0, i + 38)),
                  pl.BlockSpec((EMBED, TBLK), lambda i: (0, i + 57))],
        out_specs=pl.BlockSpec((TBLK, PAIR), lambda i: (i, 0)),
        out_shape=jax.ShapeDtypeStruct((NQUAD, PAIR), jnp.float32),
    )(tabT, tabT, tabT, tabT)


SC_TILES = SPLIT // 128
TPW = -(-SC_TILES // NW)


def _sc_transpose_body(tabT_hbm, out_hbm, slab_hi, slab_lo, otile, sem):
    wid = lax.axis_index("s") * NC + lax.axis_index("c")
    t0 = wid * TPW
    t1 = jnp.minimum((wid + 1) * TPW, SC_TILES)
    iota = lax.iota(jnp.int32, 16)

    def per_tile(t, carry):
        col_lo = pl.multiple_of(t * 128, 128)
        col_hi = pl.multiple_of(SPLIT + t * 128, 128)
        chi = pltpu.async_copy(tabT_hbm.at[:, pl.ds(col_hi, 128)], slab_hi, sem)
        clo = pltpu.async_copy(tabT_hbm.at[:, pl.ds(col_lo, 128)], slab_lo, sem)
        chi.wait()
        clo.wait()

        def per_feat(f, c2):
            for j in range(8):
                vh = slab_hi[f, pl.ds(16 * j, 16)]
                vl = slab_lo[f, pl.ds(16 * j, 16)]
                rows16 = iota + (16 * j)
                plsc.store_scatter(otile, [rows16, jnp.full((16,), f, jnp.int32)], vh)
                plsc.store_scatter(otile, [rows16, jnp.full((16,), EMBED + f, jnp.int32)], vl)
            return c2

        lax.fori_loop(0, EMBED, per_feat, 0)
        pltpu.sync_copy(otile, out_hbm.at[pl.ds(pl.multiple_of(t * 128, 128), 128)])
        return carry

    lax.fori_loop(t0, t1, per_tile, 0)


@functools.lru_cache(maxsize=None)
def _sc_transpose():
    return pl.kernel(
        _sc_transpose_body,
        out_type=jax.ShapeDtypeStruct((NPAIR2, PAIR), jnp.float32),
        mesh=plsc.VectorSubcoreMesh(core_axis_name="c", subcore_axis_name="s",
                                    num_cores=NC, num_subcores=NS),
        scratch_types=[
            pltpu.VMEM((EMBED, 128), jnp.float32),
            pltpu.VMEM((EMBED, 128), jnp.float32),
            pltpu.VMEM((128, PAIR), jnp.float32),
            pltpu.SemaphoreType.DMA,
        ],
        compiler_params=pltpu.CompilerParams(use_tc_tiling_on_sc=True,
                                             needs_layout_passes=False),
    )


def _tail_body(prev_ref, hi_ref, lo_ref, dst_ref):
    dst_ref[:, 0:EMBED] = hi_ref[...].T
    dst_ref[:, EMBED:PAIR] = lo_ref[...].T


def _tail_fixup(sc_out, tabT):
    return pl.pallas_call(
        _tail_body,
        grid=(1,),
        in_specs=[
            pl.BlockSpec(memory_space=pl.ANY),
            pl.BlockSpec((EMBED, TBLK), lambda i: (0, 2 * NSPLIT)),
            pl.BlockSpec((EMBED, TBLK), lambda i: (0, NSPLIT)),
        ],
        out_specs=pl.BlockSpec((TBLK, PAIR), lambda i: (NSPLIT, 0)),
        out_shape=jax.ShapeDtypeStruct((NPAIR2, PAIR), jnp.float32),
        input_output_aliases={0: 0},
    )(sc_out, tabT, tabT)


def _gather_body(idx_hbm, tab_hbm, out_hbm, idx_v, rows, sem):
    wid = lax.axis_index("s") * NC + lax.axis_index("c")
    base = wid * B_PER_W
    pltpu.sync_copy(idx_hbm.at[wid], idx_v)
    copies = []
    for j in range(NCHUNK):
        copies.append(pltpu.async_copy(
            tab_hbm.at[idx_v.at[j]], rows.at[pl.ds(j * CHUNK, CHUNK)], sem))
    for c in copies:
        c.wait()
    pltpu.sync_copy(rows, out_hbm.at[pl.ds(base, B_PER_W)])


@functools.lru_cache(maxsize=None)
def _sc_gather():
    return pl.kernel(
        _gather_body,
        out_type=jax.ShapeDtypeStruct((BATCH, PAIR), jnp.float32),
        mesh=plsc.VectorSubcoreMesh(core_axis_name="c", subcore_axis_name="s",
                                    num_cores=NC, num_subcores=NS),
        scratch_types=[
            pltpu.VMEM((NCHUNK, CHUNK), jnp.int32),
            pltpu.VMEM((B_PER_W, PAIR), jnp.float32),
            pltpu.SemaphoreType.DMA,
        ],
        compiler_params=pltpu.CompilerParams(use_tc_tiling_on_sc=True),
    )


ROWS_BLK = 2048
GRID = BATCH // ROWS_BLK


def _dot_t(lhs_t, rhs):
    return lax.dot_general(lhs_t, rhs, (((0,), (0,)), ((), ())),
                           preferred_element_type=jnp.float32)


def _unpack_hi(w):
    wi = lax.bitcast_convert_type(w, jnp.int32)
    return lax.bitcast_convert_type(wi & jnp.int32(-65536), jnp.float32)


def _unpack_lo(w):
    wi = lax.bitcast_convert_type(w, jnp.int32)
    return lax.bitcast_convert_type(lax.shift_left(wi, 16), jnp.float32)


def _quad_select(x4, s0, s1):
    w01 = x4[:, 0:EMBED]
    w23 = x4[:, EMBED:PAIR]
    m0 = _unpack_hi(w01) + (_unpack_lo(w01) - _unpack_hi(w01)) * s0
    m1 = _unpack_hi(w23) + (_unpack_lo(w23) - _unpack_hi(w23)) * s0
    return m0 + (m1 - m0) * s1


def _mlp_body(u4_ref, i4_ref, ft_ref, su0_ref, su1_ref, si0_ref, si1_ref,
              w1_ref, b1_ref, wrp_ref, brp_ref, rat_ref, play_ref):
    u = _quad_select(u4_ref[...], su0_ref[...], su1_ref[...])
    i = _quad_select(i4_ref[...], si0_ref[...], si1_ref[...])
    x = (jnp.dot(u, w1_ref[0:EMBED, :], preferred_element_type=jnp.float32)
         + jnp.dot(i, w1_ref[EMBED:2 * EMBED, :],
                   preferred_element_type=jnp.float32)
         + _dot_t(ft_ref[...], w1_ref[2 * EMBED:KDIM, :])
         + b1_ref[...])
    h = 0.5 * x * (1.0 + lax.erf(x * (1.0 / math.sqrt(2.0))))
    o = jnp.dot(h, wrp_ref[...], preferred_element_type=jnp.float32) + brp_ref[...]
    rat_ref[...] = jax.nn.sigmoid(o[:, 0:1])
    play_ref[...] = jnp.maximum(o[:, 1:2], 0.0)


def _mlp(u4_rows, i4_rows, fT, su0, su1, si0, si1, W1, b1, Wrp, brp,
         interpret=False):
    return pl.pallas_call(
        _mlp_body,
        grid=(GRID,),
        in_specs=[
            pl.BlockSpec((ROWS_BLK, PAIR), lambda i: (i, 0)),
            pl.BlockSpec((ROWS_BLK, PAIR), lambda i: (i, 0)),
            pl.BlockSpec((FEAT, ROWS_BLK), lambda i: (0, i)),
            pl.BlockSpec((ROWS_BLK, 1), lambda i: (i, 0)),
            pl.BlockSpec((ROWS_BLK, 1), lambda i: (i, 0)),
            pl.BlockSpec((ROWS_BLK, 1), lambda i: (i, 0)),
            pl.BlockSpec((ROWS_BLK, 1), lambda i: (i, 0)),
            pl.BlockSpec((KDIM, HIDDEN), lambda i: (0, 0)),
            pl.BlockSpec((1, HIDDEN), lambda i: (0, 0)),
            pl.BlockSpec((HIDDEN, 2), lambda i: (0, 0)),
            pl.BlockSpec((1, 2), lambda i: (0, 0)),
        ],
        out_specs=[
            pl.BlockSpec((ROWS_BLK, 1), lambda i: (i, 0)),
            pl.BlockSpec((ROWS_BLK, 1), lambda i: (i, 0)),
        ],
        out_shape=[
            jax.ShapeDtypeStruct((BATCH, 1), jnp.float32),
            jax.ShapeDtypeStruct((BATCH, 1), jnp.float32),
        ],
        interpret=interpret,
    )(u4_rows, i4_rows, fT, su0, su1, si0, si1, W1, b1, Wrp, brp)


def kernel(user_input, item_input, feature_input, user_emb, item_emb,
           W1, b1, Wr, br, Wp, bp):
    u4tab = _transpose_quad(user_emb.T)
    i4tab = _transpose_quad(item_emb.T)
    qu = jnp.minimum(user_input // QUART, 3)
    qi = jnp.minimum(item_input // QUART, 3)
    uidx = (user_input - qu * QUART).reshape(NW, NCHUNK, CHUNK)
    iidx = (item_input - qi * QUART).reshape(NW, NCHUNK, CHUNK)
    su0 = (qu & 1).astype(jnp.float32).reshape(BATCH, 1)
    su1 = (qu >> 1).astype(jnp.float32).reshape(BATCH, 1)
    si0 = (qi & 1).astype(jnp.float32).reshape(BATCH, 1)
    si1 = (qi >> 1).astype(jnp.float32).reshape(BATCH, 1)
    gather = _sc_gather()
    u4_rows = gather(uidx, u4tab)
    i4_rows = gather(iidx, i4tab)
    fT = feature_input.T
    Wrp = jnp.concatenate([Wr, Wp], axis=1)
    brp = jnp.concatenate([br, bp]).reshape(1, 2)
    rating, playtime = _mlp(u4_rows, i4_rows, fT, su0, su1, si0, si1,
                            W1, b1.reshape(1, HIDDEN), Wrp, brp)
    return (rating, playtime)


# speedup vs baseline: 3.4017x; 1.0605x over previous
import functools
import math

import jax
import jax.numpy as jnp
from jax import lax
from jax.experimental import pallas as pl
from jax.experimental.pallas import tpu as pltpu
from jax.experimental.pallas import tpu_sc as plsc

BATCH = 16384
EMBED = 64
FEAT = 64
HIDDEN = 256
KDIM = EMBED + EMBED + FEAT
PAIR = 2 * EMBED
NROWS = 1000000
NPAIR = NROWS // 2

NC = 2
NS = 16
NW = NC * NS
B_PER_W = BATCH // NW
CHUNK = 128
NCHUNK = B_PER_W // CHUNK

TBLK = 12800
NSPLIT = 39
SPLIT = NSPLIT * TBLK
NPAIR2 = NROWS - SPLIT
TGRID = -(-NPAIR2 // TBLK)


def _transpose_body(hi_ref, lo_ref, dst_ref):
    dst_ref[:, 0:EMBED] = hi_ref[...].T
    dst_ref[:, EMBED:PAIR] = lo_ref[...].T


def _transpose_pairs(tabT):
    return pl.pallas_call(
        _transpose_body,
        grid=(TGRID,),
        in_specs=[pl.BlockSpec((EMBED, TBLK), lambda i: (0, i + NSPLIT)),
                  pl.BlockSpec((EMBED, TBLK), lambda i: (0, i))],
        out_specs=pl.BlockSpec((TBLK, PAIR), lambda i: (i, 0)),
        out_shape=jax.ShapeDtypeStruct((NPAIR2, PAIR), jnp.float32),
    )(tabT, tabT)


QUART = 19 * TBLK
NQUAD = NROWS - 3 * QUART
QGRID = -(-NQUAD // TBLK)


def _pack2(a, b):
    ai = lax.bitcast_convert_type(a, jnp.int32)
    bi = lax.bitcast_convert_type(b, jnp.int32)
    w = (ai & jnp.int32(-65536)) | lax.shift_right_logical(bi, 16)
    return lax.bitcast_convert_type(w, jnp.float32)


def _transpose_quad_body(q0_ref, q1_ref, q2_ref, q3_ref, dst_ref):
    dst_ref[:, 0:EMBED] = _pack2(q0_ref[...].T, q1_ref[...].T)
    dst_ref[:, EMBED:PAIR] = _pack2(q2_ref[...].T, q3_ref[...].T)


def _transpose_quad(tabT):
    return pl.pallas_call(
        _transpose_quad_body,
        grid=(QGRID,),
        in_specs=[pl.BlockSpec((EMBED, TBLK), lambda i: (0, i)),
                  pl.BlockSpec((EMBED, TBLK), lambda i: (0, i + 19)),
                  pl.BlockSpec((EMBED, TBLK), lambda i: (0, i + 38)),
                  pl.BlockSpec((EMBED, TBLK), lambda i: (0, i + 57))],
        out_specs=pl.BlockSpec((TBLK, PAIR), lambda i: (i, 0)),
        out_shape=jax.ShapeDtypeStruct((NQUAD, PAIR), jnp.float32),
    )(tabT, tabT, tabT, tabT)


SC_TILES = SPLIT // 128
TPW = -(-SC_TILES // NW)


def _sc_transpose_body(tabT_hbm, out_hbm, slab_hi, slab_lo, otile, sem):
    wid = lax.axis_index("s") * NC + lax.axis_index("c")
    t0 = wid * TPW
    t1 = jnp.minimum((wid + 1) * TPW, SC_TILES)
    iota = lax.iota(jnp.int32, 16)

    def per_tile(t, carry):
        col_lo = pl.multiple_of(t * 128, 128)
        col_hi = pl.multiple_of(SPLIT + t * 128, 128)
        chi = pltpu.async_copy(tabT_hbm.at[:, pl.ds(col_hi, 128)], slab_hi, sem)
        clo = pltpu.async_copy(tabT_hbm.at[:, pl.ds(col_lo, 128)], slab_lo, sem)
        chi.wait()
        clo.wait()

        def per_feat(f, c2):
            for j in range(8):
                vh = slab_hi[f, pl.ds(16 * j, 16)]
                vl = slab_lo[f, pl.ds(16 * j, 16)]
                rows16 = iota + (16 * j)
                plsc.store_scatter(otile, [rows16, jnp.full((16,), f, jnp.int32)], vh)
                plsc.store_scatter(otile, [rows16, jnp.full((16,), EMBED + f, jnp.int32)], vl)
            return c2

        lax.fori_loop(0, EMBED, per_feat, 0)
        pltpu.sync_copy(otile, out_hbm.at[pl.ds(pl.multiple_of(t * 128, 128), 128)])
        return carry

    lax.fori_loop(t0, t1, per_tile, 0)


@functools.lru_cache(maxsize=None)
def _sc_transpose():
    return pl.kernel(
        _sc_transpose_body,
        out_type=jax.ShapeDtypeStruct((NPAIR2, PAIR), jnp.float32),
        mesh=plsc.VectorSubcoreMesh(core_axis_name="c", subcore_axis_name="s",
                                    num_cores=NC, num_subcores=NS),
        scratch_types=[
            pltpu.VMEM((EMBED, 128), jnp.float32),
            pltpu.VMEM((EMBED, 128), jnp.float32),
            pltpu.VMEM((128, PAIR), jnp.float32),
            pltpu.SemaphoreType.DMA,
        ],
        compiler_params=pltpu.CompilerParams(use_tc_tiling_on_sc=True,
                                             needs_layout_passes=False),
    )


def _tail_body(prev_ref, hi_ref, lo_ref, dst_ref):
    dst_ref[:, 0:EMBED] = hi_ref[...].T
    dst_ref[:, EMBED:PAIR] = lo_ref[...].T


def _tail_fixup(sc_out, tabT):
    return pl.pallas_call(
        _tail_body,
        grid=(1,),
        in_specs=[
            pl.BlockSpec(memory_space=pl.ANY),
            pl.BlockSpec((EMBED, TBLK), lambda i: (0, 2 * NSPLIT)),
            pl.BlockSpec((EMBED, TBLK), lambda i: (0, NSPLIT)),
        ],
        out_specs=pl.BlockSpec((TBLK, PAIR), lambda i: (NSPLIT, 0)),
        out_shape=jax.ShapeDtypeStruct((NPAIR2, PAIR), jnp.float32),
        input_output_aliases={0: 0},
    )(sc_out, tabT, tabT)


def _gather_body(idx_hbm, tab_hbm, out_hbm, idx_v, rows, sem):
    wid = lax.axis_index("s") * NC + lax.axis_index("c")
    base = wid * B_PER_W
    pltpu.sync_copy(idx_hbm.at[wid], idx_v)
    copies = []
    for j in range(NCHUNK):
        copies.append(pltpu.async_copy(
            tab_hbm.at[idx_v.at[j]], rows.at[pl.ds(j * CHUNK, CHUNK)], sem))
    for c in copies:
        c.wait()
    pltpu.sync_copy(rows, out_hbm.at[pl.ds(base, B_PER_W)])


@functools.lru_cache(maxsize=None)
def _sc_gather():
    return pl.kernel(
        _gather_body,
        out_type=jax.ShapeDtypeStruct((BATCH, PAIR), jnp.float32),
        mesh=plsc.VectorSubcoreMesh(core_axis_name="c", subcore_axis_name="s",
                                    num_cores=NC, num_subcores=NS),
        scratch_types=[
            pltpu.VMEM((NCHUNK, CHUNK), jnp.int32),
            pltpu.VMEM((B_PER_W, PAIR), jnp.float32),
            pltpu.SemaphoreType.DMA,
        ],
        compiler_params=pltpu.CompilerParams(use_tc_tiling_on_sc=True),
    )


ROWS_BLK = 2048
GRID = BATCH // ROWS_BLK


def _dot_t(lhs_t, rhs):
    return lax.dot_general(lhs_t, rhs, (((0,), (0,)), ((), ())),
                           preferred_element_type=jnp.float32)


def _unpack_hi(w):
    wi = lax.bitcast_convert_type(w, jnp.int32)
    return lax.bitcast_convert_type(wi & jnp.int32(-65536), jnp.float32)


def _unpack_lo(w):
    wi = lax.bitcast_convert_type(w, jnp.int32)
    return lax.bitcast_convert_type(lax.shift_left(wi, 16), jnp.float32)


def _quad_select(x4, s0, s1):
    w01 = x4[:, 0:EMBED]
    w23 = x4[:, EMBED:PAIR]
    m0 = _unpack_hi(w01) + (_unpack_lo(w01) - _unpack_hi(w01)) * s0
    m1 = _unpack_hi(w23) + (_unpack_lo(w23) - _unpack_hi(w23)) * s0
    return m0 + (m1 - m0) * s1


def _mlp_body(u4_ref, i4_ref, ft_ref, sel_ref, w1_ref, b1_ref,
              wrp_ref, brp_ref, rat_ref, play_ref):
    s = sel_ref[...]
    si1 = jnp.floor(s * 0.125)
    s = s - 8.0 * si1
    si0 = jnp.floor(s * 0.25)
    s = s - 4.0 * si0
    su1 = jnp.floor(s * 0.5)
    su0 = s - 2.0 * su1
    u = _quad_select(u4_ref[...], su0, su1)
    i = _quad_select(i4_ref[...], si0, si1)
    x = (jnp.dot(u, w1_ref[0:EMBED, :], preferred_element_type=jnp.float32)
         + jnp.dot(i, w1_ref[EMBED:2 * EMBED, :],
                   preferred_element_type=jnp.float32)
         + _dot_t(ft_ref[...], w1_ref[2 * EMBED:KDIM, :])
         + b1_ref[...])
    h = 0.5 * x * (1.0 + lax.erf(x * (1.0 / math.sqrt(2.0))))
    o_t = lax.dot_general(wrp_ref[...], h, (((0,), (1,)), ((), ())),
                          preferred_element_type=jnp.float32) + brp_ref[...]
    rat_ref[...] = jax.nn.sigmoid(o_t[0:1, :])
    play_ref[...] = jnp.maximum(o_t[1:2, :], 0.0)


def _mlp(u4_rows, i4_rows, fT, sel, W1, b1, Wrp, brp, interpret=False):
    return pl.pallas_call(
        _mlp_body,
        grid=(GRID,),
        in_specs=[
            pl.BlockSpec((ROWS_BLK, PAIR), lambda i: (i, 0)),
            pl.BlockSpec((ROWS_BLK, PAIR), lambda i: (i, 0)),
            pl.BlockSpec((FEAT, ROWS_BLK), lambda i: (0, i)),
            pl.BlockSpec((ROWS_BLK, 1), lambda i: (i, 0)),
            pl.BlockSpec((KDIM, HIDDEN), lambda i: (0, 0)),
            pl.BlockSpec((1, HIDDEN), lambda i: (0, 0)),
            pl.BlockSpec((HIDDEN, 2), lambda i: (0, 0)),
            pl.BlockSpec((2, 1), lambda i: (0, 0)),
        ],
        out_specs=[
            pl.BlockSpec((1, ROWS_BLK), lambda i: (0, i)),
            pl.BlockSpec((1, ROWS_BLK), lambda i: (0, i)),
        ],
        out_shape=[
            jax.ShapeDtypeStruct((1, BATCH), jnp.float32),
            jax.ShapeDtypeStruct((1, BATCH), jnp.float32),
        ],
        interpret=interpret,
    )(u4_rows, i4_rows, fT, sel, W1, b1, Wrp, brp)


def kernel(user_input, item_input, feature_input, user_emb, item_emb,
           W1, b1, Wr, br, Wp, bp):
    u4tab = _transpose_quad(user_emb.T)
    i4tab = _transpose_quad(item_emb.T)
    qu = jnp.minimum(user_input // QUART, 3)
    qi = jnp.minimum(item_input // QUART, 3)
    uidx = (user_input - qu * QUART).reshape(NW, NCHUNK, CHUNK)
    iidx = (item_input - qi * QUART).reshape(NW, NCHUNK, CHUNK)
    sel = (qu + 4 * qi).astype(jnp.float32).reshape(BATCH, 1)
    gather = _sc_gather()
    u4_rows = gather(uidx, u4tab)
    i4_rows = gather(iidx, i4tab)
    fT = feature_input.T
    Wrp = jnp.concatenate([Wr, Wp], axis=1)
    brp = jnp.concatenate([br, bp]).reshape(2, 1)
    rat_t, play_t = _mlp(u4_rows, i4_rows, fT, sel,
                         W1, b1.reshape(1, HIDDEN), Wrp, brp)
    return (rat_t.T, play_t.T)


# speedup vs baseline: 3.4192x; 1.0052x over previous
import functools
import math

import jax
import jax.numpy as jnp
from jax import lax
from jax.experimental import pallas as pl
from jax.experimental.pallas import tpu as pltpu
from jax.experimental.pallas import tpu_sc as plsc

BATCH = 16384
EMBED = 64
FEAT = 64
HIDDEN = 256
KDIM = EMBED + EMBED + FEAT
PAIR = 2 * EMBED
NROWS = 1000000
NPAIR = NROWS // 2

NC = 2
NS = 16
NW = NC * NS
B_PER_W = BATCH // NW
CHUNK = 128
NCHUNK = B_PER_W // CHUNK

TBLK = 12800
NSPLIT = 39
SPLIT = NSPLIT * TBLK
NPAIR2 = NROWS - SPLIT
TGRID = -(-NPAIR2 // TBLK)


def _transpose_body(hi_ref, lo_ref, dst_ref):
    dst_ref[:, 0:EMBED] = hi_ref[...].T
    dst_ref[:, EMBED:PAIR] = lo_ref[...].T


def _transpose_pairs(tabT):
    return pl.pallas_call(
        _transpose_body,
        grid=(TGRID,),
        in_specs=[pl.BlockSpec((EMBED, TBLK), lambda i: (0, i + NSPLIT)),
                  pl.BlockSpec((EMBED, TBLK), lambda i: (0, i))],
        out_specs=pl.BlockSpec((TBLK, PAIR), lambda i: (i, 0)),
        out_shape=jax.ShapeDtypeStruct((NPAIR2, PAIR), jnp.float32),
    )(tabT, tabT)


QUART = 19 * TBLK
NQUAD = NROWS - 3 * QUART
QGRID = -(-NQUAD // TBLK)


def _pack2(a, b):
    ai = lax.bitcast_convert_type(a, jnp.int32)
    bi = lax.bitcast_convert_type(b, jnp.int32)
    w = (ai & jnp.int32(-65536)) | lax.shift_right_logical(bi, 16)
    return lax.bitcast_convert_type(w, jnp.float32)


def _transpose_quad_body(q0_ref, q1_ref, q2_ref, q3_ref, dst_ref):
    dst_ref[:, 0:EMBED] = _pack2(q0_ref[...].T, q1_ref[...].T)
    dst_ref[:, EMBED:PAIR] = _pack2(q2_ref[...].T, q3_ref[...].T)


def _transpose_quad(tabT):
    return pl.pallas_call(
        _transpose_quad_body,
        grid=(QGRID,),
        in_specs=[pl.BlockSpec((EMBED, TBLK), lambda i: (0, i)),
                  pl.BlockSpec((EMBED, TBLK), lambda i: (0, i + 19)),
                  pl.BlockSpec((EMBED, TBLK), lambda i: (0, i + 38)),
                  pl.BlockSpec((EMBED, TBLK), lambda i: (0, i + 57))],
        out_specs=pl.BlockSpec((TBLK, PAIR), lambda i: (i, 0)),
        out_shape=jax.ShapeDtypeStruct((NQUAD, PAIR), jnp.float32),
    )(tabT, tabT, tabT, tabT)


SC_TILES = SPLIT // 128
TPW = -(-SC_TILES // NW)


def _sc_transpose_body(tabT_hbm, out_hbm, slab_hi, slab_lo, otile, sem):
    wid = lax.axis_index("s") * NC + lax.axis_index("c")
    t0 = wid * TPW
    t1 = jnp.minimum((wid + 1) * TPW, SC_TILES)
    iota = lax.iota(jnp.int32, 16)

    def per_tile(t, carry):
        col_lo = pl.multiple_of(t * 128, 128)
        col_hi = pl.multiple_of(SPLIT + t * 128, 128)
        chi = pltpu.async_copy(tabT_hbm.at[:, pl.ds(col_hi, 128)], slab_hi, sem)
        clo = pltpu.async_copy(tabT_hbm.at[:, pl.ds(col_lo, 128)], slab_lo, sem)
        chi.wait()
        clo.wait()

        def per_feat(f, c2):
            for j in range(8):
                vh = slab_hi[f, pl.ds(16 * j, 16)]
                vl = slab_lo[f, pl.ds(16 * j, 16)]
                rows16 = iota + (16 * j)
                plsc.store_scatter(otile, [rows16, jnp.full((16,), f, jnp.int32)], vh)
                plsc.store_scatter(otile, [rows16, jnp.full((16,), EMBED + f, jnp.int32)], vl)
            return c2

        lax.fori_loop(0, EMBED, per_feat, 0)
        pltpu.sync_copy(otile, out_hbm.at[pl.ds(pl.multiple_of(t * 128, 128), 128)])
        return carry

    lax.fori_loop(t0, t1, per_tile, 0)


@functools.lru_cache(maxsize=None)
def _sc_transpose():
    return pl.kernel(
        _sc_transpose_body,
        out_type=jax.ShapeDtypeStruct((NPAIR2, PAIR), jnp.float32),
        mesh=plsc.VectorSubcoreMesh(core_axis_name="c", subcore_axis_name="s",
                                    num_cores=NC, num_subcores=NS),
        scratch_types=[
            pltpu.VMEM((EMBED, 128), jnp.float32),
            pltpu.VMEM((EMBED, 128), jnp.float32),
            pltpu.VMEM((128, PAIR), jnp.float32),
            pltpu.SemaphoreType.DMA,
        ],
        compiler_params=pltpu.CompilerParams(use_tc_tiling_on_sc=True,
                                             needs_layout_passes=False),
    )


def _tail_body(prev_ref, hi_ref, lo_ref, dst_ref):
    dst_ref[:, 0:EMBED] = hi_ref[...].T
    dst_ref[:, EMBED:PAIR] = lo_ref[...].T


def _tail_fixup(sc_out, tabT):
    return pl.pallas_call(
        _tail_body,
        grid=(1,),
        in_specs=[
            pl.BlockSpec(memory_space=pl.ANY),
            pl.BlockSpec((EMBED, TBLK), lambda i: (0, 2 * NSPLIT)),
            pl.BlockSpec((EMBED, TBLK), lambda i: (0, NSPLIT)),
        ],
        out_specs=pl.BlockSpec((TBLK, PAIR), lambda i: (NSPLIT, 0)),
        out_shape=jax.ShapeDtypeStruct((NPAIR2, PAIR), jnp.float32),
        input_output_aliases={0: 0},
    )(sc_out, tabT, tabT)


def _gather_body(idx_hbm, tab_hbm, out_hbm, idx_v, rows, sem):
    wid = lax.axis_index("s") * NC + lax.axis_index("c")
    base = wid * B_PER_W
    pltpu.sync_copy(idx_hbm.at[wid], idx_v)
    copies = []
    for j in range(NCHUNK):
        copies.append(pltpu.async_copy(
            tab_hbm.at[idx_v.at[j]], rows.at[pl.ds(j * CHUNK, CHUNK)], sem))
    for c in copies:
        c.wait()
    pltpu.sync_copy(rows, out_hbm.at[pl.ds(base, B_PER_W)])


@functools.lru_cache(maxsize=None)
def _sc_gather():
    return pl.kernel(
        _gather_body,
        out_type=jax.ShapeDtypeStruct((BATCH, PAIR), jnp.float32),
        mesh=plsc.VectorSubcoreMesh(core_axis_name="c", subcore_axis_name="s",
                                    num_cores=NC, num_subcores=NS),
        scratch_types=[
            pltpu.VMEM((NCHUNK, CHUNK), jnp.int32),
            pltpu.VMEM((B_PER_W, PAIR), jnp.float32),
            pltpu.SemaphoreType.DMA,
        ],
        compiler_params=pltpu.CompilerParams(use_tc_tiling_on_sc=True),
    )


ROWS_BLK = 4096
GRID = BATCH // ROWS_BLK


def _dot_t(lhs_t, rhs):
    return lax.dot_general(lhs_t, rhs, (((0,), (0,)), ((), ())),
                           preferred_element_type=jnp.float32)


def _unpack_hi(w):
    wi = lax.bitcast_convert_type(w, jnp.int32)
    return lax.bitcast_convert_type(wi & jnp.int32(-65536), jnp.float32)


def _unpack_lo(w):
    wi = lax.bitcast_convert_type(w, jnp.int32)
    return lax.bitcast_convert_type(lax.shift_left(wi, 16), jnp.float32)


def _quad_select(x4, s0, s1):
    w01 = x4[:, 0:EMBED]
    w23 = x4[:, EMBED:PAIR]
    m0 = _unpack_hi(w01) + (_unpack_lo(w01) - _unpack_hi(w01)) * s0
    m1 = _unpack_hi(w23) + (_unpack_lo(w23) - _unpack_hi(w23)) * s0
    return m0 + (m1 - m0) * s1


def _mlp_body(u4_ref, i4_ref, ft_ref, sel_ref, w1_ref, b1_ref,
              wrp_ref, brp_ref, rat_ref, play_ref):
    s = sel_ref[...]
    si1 = jnp.floor(s * 0.125)
    s = s - 8.0 * si1
    si0 = jnp.floor(s * 0.25)
    s = s - 4.0 * si0
    su1 = jnp.floor(s * 0.5)
    su0 = s - 2.0 * su1
    u = _quad_select(u4_ref[...], su0, su1)
    i = _quad_select(i4_ref[...], si0, si1)
    x = (jnp.dot(u, w1_ref[0:EMBED, :], preferred_element_type=jnp.float32)
         + jnp.dot(i, w1_ref[EMBED:2 * EMBED, :],
                   preferred_element_type=jnp.float32)
         + _dot_t(ft_ref[...], w1_ref[2 * EMBED:KDIM, :])
         + b1_ref[...])
    h = 0.5 * x * (1.0 + lax.erf(x * (1.0 / math.sqrt(2.0))))
    o_t = lax.dot_general(wrp_ref[...], h, (((0,), (1,)), ((), ())),
                          preferred_element_type=jnp.float32) + brp_ref[...]
    rat_ref[...] = jax.nn.sigmoid(o_t[0:1, :])
    play_ref[...] = jnp.maximum(o_t[1:2, :], 0.0)


def _mlp(u4_rows, i4_rows, fT, sel, W1, b1, Wrp, brp, interpret=False):
    return pl.pallas_call(
        _mlp_body,
        grid=(GRID,),
        in_specs=[
            pl.BlockSpec((ROWS_BLK, PAIR), lambda i: (i, 0)),
            pl.BlockSpec((ROWS_BLK, PAIR), lambda i: (i, 0)),
            pl.BlockSpec((FEAT, ROWS_BLK), lambda i: (0, i)),
            pl.BlockSpec((ROWS_BLK, 1), lambda i: (i, 0)),
            pl.BlockSpec((KDIM, HIDDEN), lambda i: (0, 0)),
            pl.BlockSpec((1, HIDDEN), lambda i: (0, 0)),
            pl.BlockSpec((HIDDEN, 2), lambda i: (0, 0)),
            pl.BlockSpec((2, 1), lambda i: (0, 0)),
        ],
        out_specs=[
            pl.BlockSpec((1, ROWS_BLK), lambda i: (0, i)),
            pl.BlockSpec((1, ROWS_BLK), lambda i: (0, i)),
        ],
        out_shape=[
            jax.ShapeDtypeStruct((1, BATCH), jnp.float32),
            jax.ShapeDtypeStruct((1, BATCH), jnp.float32),
        ],
        interpret=interpret,
    )(u4_rows, i4_rows, fT, sel, W1, b1, Wrp, brp)


def kernel(user_input, item_input, feature_input, user_emb, item_emb,
           W1, b1, Wr, br, Wp, bp):
    u4tab = _transpose_quad(user_emb.T)
    i4tab = _transpose_quad(item_emb.T)
    qu = jnp.minimum(user_input // QUART, 3)
    qi = jnp.minimum(item_input // QUART, 3)
    uidx = (user_input - qu * QUART).reshape(NW, NCHUNK, CHUNK)
    iidx = (item_input - qi * QUART).reshape(NW, NCHUNK, CHUNK)
    sel = (qu + 4 * qi).astype(jnp.float32).reshape(BATCH, 1)
    gather = _sc_gather()
    u4_rows = gather(uidx, u4tab)
    i4_rows = gather(iidx, i4tab)
    fT = feature_input.T
    Wrp = jnp.concatenate([Wr, Wp], axis=1)
    brp = jnp.concatenate([br, bp]).reshape(2, 1)
    rat_t, play_t = _mlp(u4_rows, i4_rows, fT, sel,
                         W1, b1.reshape(1, HIDDEN), Wrp, brp)
    return (rat_t.T, play_t.T)


# speedup vs baseline: 3.4203x; 1.0003x over previous
import functools
import math

import jax
import jax.numpy as jnp
from jax import lax
from jax.experimental import pallas as pl
from jax.experimental.pallas import tpu as pltpu
from jax.experimental.pallas import tpu_sc as plsc

BATCH = 16384
EMBED = 64
FEAT = 64
HIDDEN = 256
KDIM = EMBED + EMBED + FEAT
PAIR = 2 * EMBED
NROWS = 1000000

NC = 2
NS = 16
NW = NC * NS
B_PER_W = BATCH // NW
CHUNK = 128
NCHUNK = B_PER_W // CHUNK

TBLK = 12800
QUART = 19 * TBLK
NQUAD = NROWS - 3 * QUART
QGRID = -(-NQUAD // TBLK)


def _pack2(a, b):
    ai = lax.bitcast_convert_type(a, jnp.int32)
    bi = lax.bitcast_convert_type(b, jnp.int32)
    w = (ai & jnp.int32(-65536)) | lax.shift_right_logical(bi, 16)
    return lax.bitcast_convert_type(w, jnp.float32)


def _transpose_quad_body(q0_ref, q1_ref, q2_ref, q3_ref, dst_ref):
    dst_ref[:, 0:EMBED] = _pack2(q0_ref[...].T, q1_ref[...].T)
    dst_ref[:, EMBED:PAIR] = _pack2(q2_ref[...].T, q3_ref[...].T)


def _transpose_quad(tabT):
    return pl.pallas_call(
        _transpose_quad_body,
        grid=(QGRID,),
        in_specs=[pl.BlockSpec((EMBED, TBLK), lambda i: (0, i)),
                  pl.BlockSpec((EMBED, TBLK), lambda i: (0, i + 19)),
                  pl.BlockSpec((EMBED, TBLK), lambda i: (0, i + 38)),
                  pl.BlockSpec((EMBED, TBLK), lambda i: (0, i + 57))],
        out_specs=pl.BlockSpec((TBLK, PAIR), lambda i: (i, 0)),
        out_shape=jax.ShapeDtypeStruct((NQUAD, PAIR), jnp.float32),
    )(tabT, tabT, tabT, tabT)


def _gather_body(idx_hbm, tab_hbm, out_hbm, idx_v, rows, sem):
    wid = lax.axis_index("s") * NC + lax.axis_index("c")
    base = wid * B_PER_W
    pltpu.sync_copy(idx_hbm.at[wid], idx_v)
    copies = []
    for j in range(NCHUNK):
        copies.append(pltpu.async_copy(
            tab_hbm.at[idx_v.at[j]], rows.at[pl.ds(j * CHUNK, CHUNK)], sem))
    for c in copies:
        c.wait()
    pltpu.sync_copy(rows, out_hbm.at[pl.ds(base, B_PER_W)])


@functools.lru_cache(maxsize=None)
def _sc_gather():
    return pl.kernel(
        _gather_body,
        out_type=jax.ShapeDtypeStruct((BATCH, PAIR), jnp.float32),
        mesh=plsc.VectorSubcoreMesh(core_axis_name="c", subcore_axis_name="s",
                                    num_cores=NC, num_subcores=NS),
        scratch_types=[
            pltpu.VMEM((NCHUNK, CHUNK), jnp.int32),
            pltpu.VMEM((B_PER_W, PAIR), jnp.float32),
            pltpu.SemaphoreType.DMA,
        ],
        compiler_params=pltpu.CompilerParams(use_tc_tiling_on_sc=True),
    )


ROWS_BLK = 4096
GRID = BATCH // ROWS_BLK


def _dot_t(lhs_t, rhs):
    return lax.dot_general(lhs_t, rhs, (((0,), (0,)), ((), ())),
                           preferred_element_type=jnp.float32)


def _unpack_hi(w):
    wi = lax.bitcast_convert_type(w, jnp.int32)
    return lax.bitcast_convert_type(wi & jnp.int32(-65536), jnp.float32)


def _unpack_lo(w):
    wi = lax.bitcast_convert_type(w, jnp.int32)
    return lax.bitcast_convert_type(lax.shift_left(wi, 16), jnp.float32)


def _quad_select(x4, s0, s1):
    w01 = x4[:, 0:EMBED]
    w23 = x4[:, EMBED:PAIR]
    m0 = _unpack_hi(w01) + (_unpack_lo(w01) - _unpack_hi(w01)) * s0
    m1 = _unpack_hi(w23) + (_unpack_lo(w23) - _unpack_hi(w23)) * s0
    return m0 + (m1 - m0) * s1


def _mlp_body(u4_ref, i4_ref, ft_ref, sel_ref, w1_ref, b1_ref,
              wrp_ref, brp_ref, rat_ref, play_ref):
    s = sel_ref[...]
    si1 = jnp.floor(s * 0.125)
    s = s - 8.0 * si1
    si0 = jnp.floor(s * 0.25)
    s = s - 4.0 * si0
    su1 = jnp.floor(s * 0.5)
    su0 = s - 2.0 * su1
    u = _quad_select(u4_ref[...], su0, su1)
    i = _quad_select(i4_ref[...], si0, si1)
    x = (jnp.dot(u, w1_ref[0:EMBED, :], preferred_element_type=jnp.float32)
         + jnp.dot(i, w1_ref[EMBED:2 * EMBED, :],
                   preferred_element_type=jnp.float32)
         + _dot_t(ft_ref[...], w1_ref[2 * EMBED:KDIM, :])
         + b1_ref[...])
    h = 0.5 * x * (1.0 + lax.erf(x * (1.0 / math.sqrt(2.0))))
    o_t = lax.dot_general(wrp_ref[...], h, (((0,), (1,)), ((), ())),
                          preferred_element_type=jnp.float32) + brp_ref[...]
    rat_ref[...] = jax.nn.sigmoid(o_t[0:1, :])
    play_ref[...] = jnp.maximum(o_t[1:2, :], 0.0)


def _mlp(u4_rows, i4_rows, fT, sel, W1, b1, Wrp, brp, interpret=False):
    return pl.pallas_call(
        _mlp_body,
        grid=(GRID,),
        in_specs=[
            pl.BlockSpec((ROWS_BLK, PAIR), lambda i: (i, 0)),
            pl.BlockSpec((ROWS_BLK, PAIR), lambda i: (i, 0)),
            pl.BlockSpec((FEAT, ROWS_BLK), lambda i: (0, i)),
            pl.BlockSpec((ROWS_BLK, 1), lambda i: (i, 0)),
            pl.BlockSpec((KDIM, HIDDEN), lambda i: (0, 0)),
            pl.BlockSpec((1, HIDDEN), lambda i: (0, 0)),
            pl.BlockSpec((HIDDEN, 2), lambda i: (0, 0)),
            pl.BlockSpec((2, 1), lambda i: (0, 0)),
        ],
        out_specs=[
            pl.BlockSpec((1, ROWS_BLK), lambda i: (0, i)),
            pl.BlockSpec((1, ROWS_BLK), lambda i: (0, i)),
        ],
        out_shape=[
            jax.ShapeDtypeStruct((1, BATCH), jnp.float32),
            jax.ShapeDtypeStruct((1, BATCH), jnp.float32),
        ],
        interpret=interpret,
    )(u4_rows, i4_rows, fT, sel, W1, b1, Wrp, brp)


def kernel(user_input, item_input, feature_input, user_emb, item_emb,
           W1, b1, Wr, br, Wp, bp):
    u4tab = _transpose_quad(user_emb.T)
    i4tab = _transpose_quad(item_emb.T)
    qu = jnp.minimum(user_input // QUART, 3)
    qi = jnp.minimum(item_input // QUART, 3)
    uidx = (user_input - qu * QUART).reshape(NW, NCHUNK, CHUNK)
    iidx = (item_input - qi * QUART).reshape(NW, NCHUNK, CHUNK)
    sel = (qu + 4 * qi).astype(jnp.float32).reshape(BATCH, 1)
    gather = _sc_gather()
    u4_rows = gather(uidx, u4tab)
    i4_rows = gather(iidx, i4tab)
    fT = feature_input.T
    Wrp = jnp.concatenate([Wr, Wp], axis=1)
    brp = jnp.concatenate([br, bp]).reshape(2, 1)
    rat_t, play_t = _mlp(u4_rows, i4_rows, fT, sel,
                         W1, b1.reshape(1, HIDDEN), Wrp, brp)
    return (rat_t.T, play_t.T)
